# baseline (speedup 1.0000x reference)
import functools

import jax
import jax.numpy as jnp
from jax import lax
from jax.experimental import pallas as pl
from jax.experimental.pallas import tpu as pltpu
from jax.experimental.pallas import tpu_sc as plsc

D = 64
TD = 128
B = 16384
NC, NS = 2, 16
NW = NC * NS
BPW = B // NW
CH = 128
NP = BPW // CH


def _dot_kernel(emb_hbm, vi_hbm, vj_hbm, out_hbm,
                raw_i, raw_j, pk_i, pk_j,
                ra0, rb0, ra1, rb1, out_v, sem0, sem1):
    wid = lax.axis_index("s") * NC + lax.axis_index("c")
    base = wid * BPW

    cps = []
    for j in range(NP):
        cps.append(pltpu.async_copy(
            vi_hbm.at[pl.ds(base + j * CH, CH)], raw_i.at[j], sem0))
        cps.append(pltpu.async_copy(
            vj_hbm.at[pl.ds(base + j * CH, CH)], raw_j.at[j], sem0))
    for c in cps:
        c.wait()

    for j in range(NP):
        for k in range(CH // 16):
            s = pl.ds(k * 16, 16)
            pk_i[j, s] = raw_i[j, s] >> 1
            pk_j[j, s] = raw_j[j, s] >> 1

    rows_a = (ra0, ra1)
    rows_b = (rb0, rb1)
    sems = (sem0, sem1)

    def fire(p):
        s = p % 2
        return (pltpu.async_copy(emb_hbm.at[pk_i.at[p]], rows_a[s], sems[s]),
                pltpu.async_copy(emb_hbm.at[pk_j.at[p]], rows_b[s], sems[s]))

    iota = lax.iota(jnp.int32, 16)
    pending = {0: fire(0)}
    for p in range(NP):
        if p + 1 < NP:
            pending[p + 1] = fire(p + 1)
        for h in pending.pop(p):
            h.wait()
        ra, rb = rows_a[p % 2], rows_b[p % 2]

        def block(bi, carry, _p=p, _ra=ra, _rb=rb):
            r0 = bi * 16
            va = raw_i[_p, pl.ds(r0, 16)]
            vb = raw_j[_p, pl.ds(r0, 16)]
            ca = (va & 1) << 6
            cb = (vb & 1) << 6
            rowv = r0 + iota
            acc = None
            for d in range(D):
                a = plsc.load_gather(_ra, [rowv, ca + d])
                b = plsc.load_gather(_rb, [rowv, cb + d])
                acc = a * b if acc is None else acc + a * b
            out_v[pl.ds(_p * CH + r0, 16)] = acc
            return carry

        lax.fori_loop(0, CH // 16, block, 0)

    pltpu.sync_copy(out_v, out_hbm.at[pl.ds(base, BPW)])


@jax.jit
def _run(emb2, v_i, v_j):
    mesh = plsc.VectorSubcoreMesh(core_axis_name="c", subcore_axis_name="s")
    k = functools.partial(
        pl.kernel,
        out_type=jax.ShapeDtypeStruct((B,), jnp.float32),
        mesh=mesh,
        scratch_types=[
            pltpu.VMEM((NP, CH), jnp.int32),
            pltpu.VMEM((NP, CH), jnp.int32),
            pltpu.VMEM((NP, CH), jnp.int32),
            pltpu.VMEM((NP, CH), jnp.int32),
            pltpu.VMEM((CH, TD), jnp.float32),
            pltpu.VMEM((CH, TD), jnp.float32),
            pltpu.VMEM((CH, TD), jnp.float32),
            pltpu.VMEM((CH, TD), jnp.float32),
            pltpu.VMEM((BPW,), jnp.float32),
            pltpu.SemaphoreType.DMA,
            pltpu.SemaphoreType.DMA,
        ],
        compiler_params=pltpu.CompilerParams(needs_layout_passes=False),
    )(_dot_kernel)
    return k(emb2, v_i, v_j)


def kernel(v_i, v_j, first_emb, second_emb, context_emb):
    del second_emb, context_emb
    v_i = v_i.astype(jnp.int32)
    v_j = v_j.astype(jnp.int32)
    emb2 = first_emb.reshape(first_emb.shape[0] // 2, TD)
    out = _run(emb2, v_i, v_j)
    return out.reshape(B, 1)
